# baseline (speedup 1.0000x reference)
import functools

import jax
import jax.numpy as jnp
from jax import lax
from jax.experimental import pallas as pl
from jax.experimental.pallas import tpu as pltpu
from jax.experimental.pallas import tpu_sc as plsc

B, P, C = 2, 12000, 64
NXY = 512 * 512
NQ = 8
CHUNK = NXY // NQ
LANES = 16
NC, NS = 2, 16
C_PER_TEC = C // NS
NY = 512
ROWS_PER_CHUNK = CHUNK // NY
ZU = NY // LANES


def _sc_body(flat_hbm, xt_hbm, out_hbm, idx_buf, x_buf, sel_off, sel_p,
             chunk0, chunk1, sem0, sem1):
    wid = lax.axis_index("s") * NC + lax.axis_index("c")
    b = wid // NS
    c0 = (wid % NS) * C_PER_TEC
    lane = jnp.arange(LANES, dtype=jnp.int32)
    zeros16 = jnp.zeros((LANES,), jnp.float32)

    pltpu.sync_copy(flat_hbm.at[b], idx_buf)

    starts = []
    cnt = jnp.int32(0)
    for q in range(NQ):
        starts.append(cnt)
        base = q * CHUNK

        def cbody(i, cnt, base=base):
            sl = pl.ds(i * LANES, LANES)
            idx16 = idx_buf[sl]
            m = (idx16 >= base) & (idx16 < base + CHUNK)
            plsc.store_compressed(sel_off.at[pl.ds(cnt, LANES)],
                                  idx16 - base, mask=m)
            plsc.store_compressed(sel_p.at[pl.ds(cnt, LANES)],
                                  i * LANES + lane, mask=m)
            return cnt + plsc.all_reduce_population_count(m)[0]

        cnt = lax.fori_loop(0, P // LANES, cbody, cnt)
    starts.append(cnt)
    starts = [jnp.minimum(s, 0) for s in starts]

    bufs = (chunk0, chunk1)
    sems = (sem0, sem1)
    copies = [None, None]
    for ci in range(C_PER_TEC):
        c = c0 + ci
        pltpu.sync_copy(xt_hbm.at[b * C + c], x_buf)
        for q in range(NQ):
            bi = (ci * NQ + q) % 2
            buf = bufs[bi]
            if copies[bi] is not None:
                copies[bi].wait()

            def zbody(i, carry, buf=buf):
                for k in range(ZU):
                    buf[i, pl.ds(k * LANES, LANES)] = zeros16
                return carry

            lax.fori_loop(0, ROWS_PER_CHUNK, zbody, 0)

            s_q, e_q = starts[q], starts[q + 1]

            def sbody(j, carry, s_q=s_q, e_q=e_q, buf=buf):
                pos = s_q + j * LANES
                sl = pl.ds(pos, LANES)
                m = (pos + lane) < e_q
                offc = jnp.where(m, sel_off[sl], 0)
                pc = jnp.where(m, sel_p[sl], 0)
                xv = plsc.load_gather(x_buf, [pc])
                plsc.addupdate_scatter(buf, [offc >> 9, offc & 511], xv,
                                       mask=m)
                return carry

            ntrip = (e_q - s_q + (LANES - 1)) // LANES
            lax.fori_loop(0, ntrip, sbody, 0)

            copies[bi] = pltpu.async_copy(
                buf, out_hbm.at[b, c, pl.ds(q * ROWS_PER_CHUNK,
                                            ROWS_PER_CHUNK)], sems[bi])
    copies[0].wait()
    copies[1].wait()


@functools.partial(
    pl.kernel,
    out_type=jax.ShapeDtypeStruct((B, C, 512, NY), jnp.float32),
    mesh=plsc.VectorSubcoreMesh(
        core_axis_name="c", subcore_axis_name="s",
        num_cores=NC, num_subcores=NS),
    scratch_types=[
        pltpu.VMEM((P,), jnp.int32),
        pltpu.VMEM((P,), jnp.float32),
        pltpu.VMEM((P + LANES,), jnp.int32),
        pltpu.VMEM((P + LANES,), jnp.int32),
        pltpu.VMEM((ROWS_PER_CHUNK, NY), jnp.float32),
        pltpu.VMEM((ROWS_PER_CHUNK, NY), jnp.float32),
        pltpu.SemaphoreType.DMA,
        pltpu.SemaphoreType.DMA,
    ],
    compiler_params=pltpu.CompilerParams(needs_layout_passes=False),
)
def _scatter_planes(flat_hbm, xt_hbm, out_hbm, idx_buf, x_buf, sel_off,
                    sel_p, chunk0, chunk1, sem0, sem1):
    _sc_body(flat_hbm, xt_hbm, out_hbm, idx_buf, x_buf, sel_off, sel_p,
             chunk0, chunk1, sem0, sem1)


def _tr_body(x_ref, o_ref):
    o_ref[...] = jnp.transpose(x_ref[...], (1, 0))


_transpose_x = pl.pallas_call(
    _tr_body,
    grid=(B,),
    in_specs=[pl.BlockSpec((None, P, C), lambda i: (i, 0, 0))],
    out_specs=pl.BlockSpec((None, C, P), lambda i: (i, 0, 0)),
    out_shape=jax.ShapeDtypeStruct((B, C, P), jnp.float32),
)


def kernel(x, indices):
    flat = indices[:, :, 0] * 512 + indices[:, :, 1]
    xt = _transpose_x(x).reshape(B * C, P)
    return _scatter_planes(flat, xt)


# speedup vs baseline: 1.2097x; 1.2097x over previous
import functools

import jax
import jax.numpy as jnp
from jax import lax
from jax.experimental import pallas as pl
from jax.experimental.pallas import tpu as pltpu
from jax.experimental.pallas import tpu_sc as plsc

B, P, C = 2, 12000, 64
NXY = 512 * 512
NQ = 8
CHUNK = NXY // NQ
LANES = 16
NC, NS = 2, 16
NY = 512
ROWS_PER_CHUNK = CHUNK // NY
ZU = NY // LANES
C_PER_TEC = 32


def _sc_body(flat_hbm, xt_hbm, out_hbm, idx_buf, sel_off, sel_p,
             x0, x1, chunk0, chunk1, xsem0, xsem1, sem0, sem1):
    wid = lax.axis_index("s") * NC + lax.axis_index("c")
    b = wid // NS
    rem = wid % NS
    q = rem % NQ
    ch0 = (rem // NQ) * C_PER_TEC
    base = q * CHUNK
    lane = jnp.arange(LANES, dtype=jnp.int32)
    zeros16 = jnp.zeros((LANES,), jnp.float32)

    pltpu.sync_copy(flat_hbm.at[b], idx_buf)

    def cbody(i, cnt):
        sl = pl.ds(i * LANES, LANES)
        idx16 = idx_buf[sl] - base
        m = (idx16 >= 0) & (idx16 < CHUNK)
        plsc.store_compressed(sel_off.at[pl.ds(cnt, LANES)], idx16, mask=m)
        plsc.store_compressed(sel_p.at[pl.ds(cnt, LANES)],
                              i * LANES + lane, mask=m)
        return cnt + plsc.all_reduce_population_count(m)[0]

    cnt = lax.fori_loop(0, P // LANES, cbody, jnp.int32(0))
    nfull = cnt // LANES
    ntail = cnt - nfull * LANES

    xbufs = (x0, x1)
    xsems = (xsem0, xsem1)
    bufs = (chunk0, chunk1)
    sems = (sem0, sem1)
    copies = [None, None]
    xcopy = pltpu.async_copy(xt_hbm.at[b * C + ch0], x0, xsem0)
    for ci in range(C_PER_TEC):
        c = ch0 + ci
        xbuf = xbufs[ci % 2]
        xcopy.wait()
        if ci + 1 < C_PER_TEC:
            xcopy = pltpu.async_copy(xt_hbm.at[b * C + c + 1],
                                     xbufs[(ci + 1) % 2],
                                     xsems[(ci + 1) % 2])
        buf = bufs[ci % 2]
        if copies[ci % 2] is not None:
            copies[ci % 2].wait()

        def zbody(i, carry, buf=buf):
            for k in range(ZU):
                buf[i, pl.ds(k * LANES, LANES)] = zeros16
            return carry

        lax.fori_loop(0, ROWS_PER_CHUNK, zbody, 0)

        def sbody(j, carry, buf=buf, xbuf=xbuf):
            sl = pl.ds(j * LANES, LANES)
            offc = sel_off[sl]
            pc = sel_p[sl]
            xv = plsc.load_gather(xbuf, [pc])
            plsc.addupdate_scatter(buf, [offc >> 9, offc & 511], xv)
            return carry

        lax.fori_loop(0, nfull, sbody, 0)

        pos = nfull * LANES
        m = lane < ntail
        sl = pl.ds(pos, LANES)
        offc = jnp.where(m, sel_off[sl], 0)
        pc = jnp.where(m, sel_p[sl], 0)
        xv = plsc.load_gather(xbuf, [pc])
        plsc.addupdate_scatter(buf, [offc >> 9, offc & 511], xv, mask=m)

        copies[ci % 2] = pltpu.async_copy(
            buf, out_hbm.at[b, c, pl.ds(q * ROWS_PER_CHUNK,
                                        ROWS_PER_CHUNK)], sems[ci % 2])
    copies[0].wait()
    copies[1].wait()


@functools.partial(
    pl.kernel,
    out_type=jax.ShapeDtypeStruct((B, C, 512, NY), jnp.float32),
    mesh=plsc.VectorSubcoreMesh(
        core_axis_name="c", subcore_axis_name="s",
        num_cores=NC, num_subcores=NS),
    scratch_types=[
        pltpu.VMEM((P,), jnp.int32),
        pltpu.VMEM((P + LANES,), jnp.int32),
        pltpu.VMEM((P + LANES,), jnp.int32),
        pltpu.VMEM((P,), jnp.float32),
        pltpu.VMEM((P,), jnp.float32),
        pltpu.VMEM((ROWS_PER_CHUNK, NY), jnp.float32),
        pltpu.VMEM((ROWS_PER_CHUNK, NY), jnp.float32),
        pltpu.SemaphoreType.DMA,
        pltpu.SemaphoreType.DMA,
        pltpu.SemaphoreType.DMA,
        pltpu.SemaphoreType.DMA,
    ],
    compiler_params=pltpu.CompilerParams(needs_layout_passes=False),
)
def _scatter_planes(flat_hbm, xt_hbm, out_hbm, idx_buf, sel_off, sel_p,
                    x0, x1, chunk0, chunk1, xsem0, xsem1, sem0, sem1):
    _sc_body(flat_hbm, xt_hbm, out_hbm, idx_buf, sel_off, sel_p,
             x0, x1, chunk0, chunk1, xsem0, xsem1, sem0, sem1)


def _tr_body(x_ref, o_ref):
    o_ref[...] = jnp.transpose(x_ref[...], (1, 0))


_transpose_x = pl.pallas_call(
    _tr_body,
    grid=(B,),
    in_specs=[pl.BlockSpec((None, P, C), lambda i: (i, 0, 0))],
    out_specs=pl.BlockSpec((None, C, P), lambda i: (i, 0, 0)),
    out_shape=jax.ShapeDtypeStruct((B, C, P), jnp.float32),
)


def kernel(x, indices):
    flat = indices[:, :, 0] * 512 + indices[:, :, 1]
    xt = _transpose_x(x).reshape(B * C, P)
    return _scatter_planes(flat, xt)


# speedup vs baseline: 1.2527x; 1.0355x over previous
import functools

import jax
import jax.numpy as jnp
from jax import lax
from jax.experimental import pallas as pl
from jax.experimental.pallas import tpu as pltpu
from jax.experimental.pallas import tpu_sc as plsc

B, P, C = 2, 12000, 64
NXY = 512 * 512
NQ = 8
CHUNK = NXY // NQ
LANES = 16
NC, NS = 2, 16
NY = 512
ROWS_PER_CHUNK = CHUNK // NY
ZU = NY // LANES
C_PER_TEC = 32


def _sc_body(flat_hbm, xt_hbm, out_hbm, idx_buf, sel_off, sel_p,
             x0, x1, chunk0, chunk1, xsem0, xsem1, sem0, sem1):
    wid = lax.axis_index("s") * NC + lax.axis_index("c")
    b = wid // NS
    rem = wid % NS
    q = rem % NQ
    ch0 = (rem // NQ) * C_PER_TEC
    base = q * CHUNK
    lane = jnp.arange(LANES, dtype=jnp.int32)
    zeros16 = jnp.zeros((LANES,), jnp.float32)

    pltpu.sync_copy(flat_hbm.at[b], idx_buf)

    def cbody(i, cnt):
        sl = pl.ds(i * LANES, LANES)
        idx16 = idx_buf[sl] - base
        m = (idx16 >= 0) & (idx16 < CHUNK)
        plsc.store_compressed(sel_off.at[pl.ds(cnt, LANES)], idx16, mask=m)
        plsc.store_compressed(sel_p.at[pl.ds(cnt, LANES)],
                              i * LANES + lane, mask=m)
        return cnt + plsc.all_reduce_population_count(m)[0]

    cnt = lax.fori_loop(0, P // LANES, cbody, jnp.int32(0))
    nfull = cnt // LANES
    ntail = cnt - nfull * LANES

    xbufs = (x0, x1)
    xsems = (xsem0, xsem1)
    bufs = (chunk0, chunk1)
    sems = (sem0, sem1)
    copies = [None, None]
    xcopy = pltpu.async_copy(xt_hbm.at[b * C + ch0], x0, xsem0)

    for buf in bufs:
        def zbody(i, carry, buf=buf):
            for k in range(ZU):
                buf[i, pl.ds(k * LANES, LANES)] = zeros16
            return carry

        lax.fori_loop(0, ROWS_PER_CHUNK, zbody, 0)

    tpos = nfull * LANES
    tm = lane < ntail
    tsl = pl.ds(tpos, LANES)
    toffc = jnp.where(tm, sel_off[tsl], 0)
    tpc = jnp.where(tm, sel_p[tsl], 0)

    for ci in range(C_PER_TEC):
        c = ch0 + ci
        xbuf = xbufs[ci % 2]
        xcopy.wait()
        if ci + 1 < C_PER_TEC:
            xcopy = pltpu.async_copy(xt_hbm.at[b * C + c + 1],
                                     xbufs[(ci + 1) % 2],
                                     xsems[(ci + 1) % 2])
        buf = bufs[ci % 2]
        if copies[ci % 2] is not None:
            copies[ci % 2].wait()

            def rbody(j, carry, buf=buf):
                offc = sel_off[pl.ds(j * LANES, LANES)]
                plsc.store_scatter(buf, [offc >> 9, offc & 511], zeros16)
                return carry

            lax.fori_loop(0, nfull, rbody, 0)
            plsc.store_scatter(buf, [toffc >> 9, toffc & 511], zeros16,
                               mask=tm)

        def sbody(j, carry, buf=buf, xbuf=xbuf):
            sl = pl.ds(j * LANES, LANES)
            offc = sel_off[sl]
            pc = sel_p[sl]
            xv = plsc.load_gather(xbuf, [pc])
            plsc.addupdate_scatter(buf, [offc >> 9, offc & 511], xv)
            return carry

        lax.fori_loop(0, nfull, sbody, 0)

        xv = plsc.load_gather(xbuf, [tpc])
        plsc.addupdate_scatter(buf, [toffc >> 9, toffc & 511], xv, mask=tm)

        copies[ci % 2] = pltpu.async_copy(
            buf, out_hbm.at[b, c, pl.ds(q * ROWS_PER_CHUNK,
                                        ROWS_PER_CHUNK)], sems[ci % 2])
    copies[0].wait()
    copies[1].wait()


@functools.partial(
    pl.kernel,
    out_type=jax.ShapeDtypeStruct((B, C, 512, NY), jnp.float32),
    mesh=plsc.VectorSubcoreMesh(
        core_axis_name="c", subcore_axis_name="s",
        num_cores=NC, num_subcores=NS),
    scratch_types=[
        pltpu.VMEM((P,), jnp.int32),
        pltpu.VMEM((P + LANES,), jnp.int32),
        pltpu.VMEM((P + LANES,), jnp.int32),
        pltpu.VMEM((P,), jnp.float32),
        pltpu.VMEM((P,), jnp.float32),
        pltpu.VMEM((ROWS_PER_CHUNK, NY), jnp.float32),
        pltpu.VMEM((ROWS_PER_CHUNK, NY), jnp.float32),
        pltpu.SemaphoreType.DMA,
        pltpu.SemaphoreType.DMA,
        pltpu.SemaphoreType.DMA,
        pltpu.SemaphoreType.DMA,
    ],
    compiler_params=pltpu.CompilerParams(needs_layout_passes=False),
)
def _scatter_planes(flat_hbm, xt_hbm, out_hbm, idx_buf, sel_off, sel_p,
                    x0, x1, chunk0, chunk1, xsem0, xsem1, sem0, sem1):
    _sc_body(flat_hbm, xt_hbm, out_hbm, idx_buf, sel_off, sel_p,
             x0, x1, chunk0, chunk1, xsem0, xsem1, sem0, sem1)


def _tr_body(x_ref, o_ref):
    o_ref[...] = jnp.transpose(x_ref[...], (1, 0))


_transpose_x = pl.pallas_call(
    _tr_body,
    grid=(B,),
    in_specs=[pl.BlockSpec((None, P, C), lambda i: (i, 0, 0))],
    out_specs=pl.BlockSpec((C, P), lambda i: (i, 0)),
    out_shape=jax.ShapeDtypeStruct((B * C, P), jnp.float32),
)


def kernel(x, indices):
    flat = indices[:, :, 0] * 512 + indices[:, :, 1]
    xt = _transpose_x(x)
    return _scatter_planes(flat, xt)


# speedup vs baseline: 1.4349x; 1.1454x over previous
import functools

import jax
import jax.numpy as jnp
from jax import lax
from jax.experimental import pallas as pl
from jax.experimental.pallas import tpu as pltpu
from jax.experimental.pallas import tpu_sc as plsc

B, P, C = 2, 12000, 64
NXY = 512 * 512
NQ = 8
CHUNK = NXY // NQ
LANES = 16
NC, NS = 2, 16
NY = 512
ROWS_PER_CHUNK = CHUNK // NY
ZU = NY // LANES
C_PER_TEC = 32


def _sc_body(flat_hbm, xt_hbm, out_hbm, idx_buf, sel_off, sel_p,
             x0, x1, chunk0, chunk1, xsem0, xsem1, sem0, sem1):
    wid = lax.axis_index("s") * NC + lax.axis_index("c")
    b = wid // NS
    rem = wid % NS
    q = rem % NQ
    ch0 = (rem // NQ) * C_PER_TEC
    base = q * CHUNK
    lane = jnp.arange(LANES, dtype=jnp.int32)
    zeros16 = jnp.zeros((LANES,), jnp.float32)

    pltpu.sync_copy(flat_hbm.at[b], idx_buf)

    def cbody(i, cnt):
        sl = pl.ds(i * LANES, LANES)
        idx16 = idx_buf[sl] - base
        m = (idx16 >= 0) & (idx16 < CHUNK)
        plsc.store_compressed(sel_off.at[pl.ds(cnt, LANES)], idx16, mask=m)
        plsc.store_compressed(sel_p.at[pl.ds(cnt, LANES)],
                              i * LANES + lane, mask=m)
        return cnt + plsc.all_reduce_population_count(m)[0]

    cnt = lax.fori_loop(0, P // LANES, cbody, jnp.int32(0))
    nfull = cnt // LANES
    ntail = cnt - nfull * LANES

    xbufs = (x0, x1)
    xsems = (xsem0, xsem1)
    bufs = (chunk0, chunk1)
    sems = (sem0, sem1)
    copies = [None, None]
    xcopy = pltpu.async_copy(xt_hbm.at[b * C + ch0], x0, xsem0)

    for buf in bufs:
        def zbody(i, carry, buf=buf):
            for k in range(ZU):
                buf[i, pl.ds(k * LANES, LANES)] = zeros16
            return carry

        lax.fori_loop(0, ROWS_PER_CHUNK, zbody, 0)

    tpos = nfull * LANES
    tm = lane < ntail
    tsl = pl.ds(tpos, LANES)
    toffc = jnp.where(tm, sel_off[tsl], 0)
    tpc = jnp.where(tm, sel_p[tsl], 0)

    for ci in range(C_PER_TEC):
        c = ch0 + ci
        xbuf = xbufs[ci % 2]
        xcopy.wait()
        if ci + 1 < C_PER_TEC:
            xcopy = pltpu.async_copy(xt_hbm.at[b * C + c + 1],
                                     xbufs[(ci + 1) % 2],
                                     xsems[(ci + 1) % 2])
        buf = bufs[ci % 2]
        if copies[ci % 2] is not None:
            copies[ci % 2].wait()

            def rbody(j, carry, buf=buf):
                offc = sel_off[pl.ds(j * LANES, LANES)]
                plsc.store_scatter(buf, [offc >> 9, offc & 511], zeros16)
                return carry

            lax.fori_loop(0, nfull, rbody, 0)
            plsc.store_scatter(buf, [toffc >> 9, toffc & 511], zeros16,
                               mask=tm)

        def sbody(j, carry, buf=buf, xbuf=xbuf):
            sl = pl.ds(j * LANES, LANES)
            offc = sel_off[sl]
            pc = sel_p[sl]
            xv = plsc.load_gather(xbuf, [pc])
            plsc.addupdate_scatter(buf, [offc >> 9, offc & 511], xv)
            return carry

        lax.fori_loop(0, nfull, sbody, 0)

        xv = plsc.load_gather(xbuf, [tpc])
        plsc.addupdate_scatter(buf, [toffc >> 9, toffc & 511], xv, mask=tm)

        copies[ci % 2] = pltpu.async_copy(
            buf, out_hbm.at[b, c, pl.ds(q * ROWS_PER_CHUNK,
                                        ROWS_PER_CHUNK)], sems[ci % 2])
    copies[0].wait()
    copies[1].wait()


@functools.partial(
    pl.kernel,
    out_type=jax.ShapeDtypeStruct((B, C, 512, NY), jnp.float32),
    mesh=plsc.VectorSubcoreMesh(
        core_axis_name="c", subcore_axis_name="s",
        num_cores=NC, num_subcores=NS),
    scratch_types=[
        pltpu.VMEM((P,), jnp.int32),
        pltpu.VMEM((P + LANES,), jnp.int32),
        pltpu.VMEM((P + LANES,), jnp.int32),
        pltpu.VMEM((P,), jnp.float32),
        pltpu.VMEM((P,), jnp.float32),
        pltpu.VMEM((ROWS_PER_CHUNK, NY), jnp.float32),
        pltpu.VMEM((ROWS_PER_CHUNK, NY), jnp.float32),
        pltpu.SemaphoreType.DMA,
        pltpu.SemaphoreType.DMA,
        pltpu.SemaphoreType.DMA,
        pltpu.SemaphoreType.DMA,
    ],
    compiler_params=pltpu.CompilerParams(needs_layout_passes=False),
)
def _scatter_planes(flat_hbm, xt_hbm, out_hbm, idx_buf, sel_off, sel_p,
                    x0, x1, chunk0, chunk1, xsem0, xsem1, sem0, sem1):
    _sc_body(flat_hbm, xt_hbm, out_hbm, idx_buf, sel_off, sel_p,
             x0, x1, chunk0, chunk1, xsem0, xsem1, sem0, sem1)


def _tr_body(x_ref, o_ref):
    o_ref[...] = jnp.transpose(x_ref[...], (1, 0))


_transpose_x = pl.pallas_call(
    _tr_body,
    grid=(B,),
    in_specs=[pl.BlockSpec((None, P, C), lambda i: (i, 0, 0))],
    out_specs=pl.BlockSpec((C, P), lambda i: (i, 0)),
    out_shape=jax.ShapeDtypeStruct((B * C, P), jnp.float32),
)


def kernel(x, indices):
    flat = indices[:, :, 0] * 512 + indices[:, :, 1]
    xt = jnp.transpose(x, (0, 2, 1)).reshape(B * C, P)
    return _scatter_planes(flat, xt)


# speedup vs baseline: 1.4357x; 1.0006x over previous
import functools

import jax
import jax.numpy as jnp
from jax import lax
from jax.experimental import pallas as pl
from jax.experimental.pallas import tpu as pltpu
from jax.experimental.pallas import tpu_sc as plsc

B, P, C = 2, 12000, 64
NXY = 512 * 512
NQ = 8
CHUNK = NXY // NQ
LANES = 16
NC, NS = 2, 16
NY = 512
ROWS_PER_CHUNK = CHUNK // NY
ZU = NY // LANES
C_PER_TEC = 32


def _sc_body(flat_hbm, xt_hbm, out_hbm, idx_buf, sel_off, sel_p,
             x0, x1, chunk0, chunk1, xsem0, xsem1, sem0, sem1):
    wid = lax.axis_index("s") * NC + lax.axis_index("c")
    b = wid // NS
    rem = wid % NS
    q = rem % NQ
    ch0 = (rem // NQ) * C_PER_TEC
    base = q * CHUNK
    lane = jnp.arange(LANES, dtype=jnp.int32)
    zeros16 = jnp.zeros((LANES,), jnp.float32)

    pltpu.sync_copy(flat_hbm.at[b], idx_buf)

    def cbody(i, cnt):
        sl = pl.ds(i * LANES, LANES)
        idx16 = idx_buf[sl] - base
        m = (idx16 >= 0) & (idx16 < CHUNK)
        plsc.store_compressed(sel_off.at[pl.ds(cnt, LANES)], idx16, mask=m)
        plsc.store_compressed(sel_p.at[pl.ds(cnt, LANES)],
                              i * LANES + lane, mask=m)
        return cnt + plsc.all_reduce_population_count(m)[0]

    cnt = lax.fori_loop(0, P // LANES, cbody, jnp.int32(0))
    nfull = cnt // LANES
    ntail = cnt - nfull * LANES

    xbufs = (x0, x1)
    xsems = (xsem0, xsem1)
    bufs = (chunk0, chunk1)
    sems = (sem0, sem1)
    copies = [None, None]
    xcopy = pltpu.async_copy(xt_hbm.at[b * C + ch0], x0, xsem0)

    for buf in bufs:
        def zbody(i, carry, buf=buf):
            for k in range(ZU):
                buf[i, pl.ds(k * LANES, LANES)] = zeros16
            return carry

        lax.fori_loop(0, ROWS_PER_CHUNK, zbody, 0)

    tpos = nfull * LANES
    tm = lane < ntail
    tsl = pl.ds(tpos, LANES)
    toffc = jnp.where(tm, sel_off[tsl], 0)
    tpc = jnp.where(tm, sel_p[tsl], 0)

    for ci in range(C_PER_TEC):
        c = ch0 + ci
        xbuf = xbufs[ci % 2]
        xcopy.wait()
        if ci + 1 < C_PER_TEC:
            xcopy = pltpu.async_copy(xt_hbm.at[b * C + c + 1],
                                     xbufs[(ci + 1) % 2],
                                     xsems[(ci + 1) % 2])
        buf = bufs[ci % 2]
        if copies[ci % 2] is not None:
            copies[ci % 2].wait()

            def rbody(j, carry, buf=buf):
                offc = sel_off[pl.ds(j * LANES, LANES)]
                plsc.store_scatter(buf, [offc >> 9, offc & 511], zeros16)
                return carry

            lax.fori_loop(0, nfull, rbody, 0)
            plsc.store_scatter(buf, [toffc >> 9, toffc & 511], zeros16,
                               mask=tm)

        def sbody(j, carry, buf=buf, xbuf=xbuf):
            sl = pl.ds(j * LANES, LANES)
            offc = sel_off[sl]
            pc = sel_p[sl]
            xv = plsc.load_gather(xbuf, [pc])
            plsc.addupdate_scatter(buf, [offc >> 9, offc & 511], xv)
            return carry

        lax.fori_loop(0, nfull, sbody, 0)

        xv = plsc.load_gather(xbuf, [tpc])
        plsc.addupdate_scatter(buf, [toffc >> 9, toffc & 511], xv, mask=tm)

        copies[ci % 2] = pltpu.async_copy(
            buf, out_hbm.at[b, c, pl.ds(q * ROWS_PER_CHUNK,
                                        ROWS_PER_CHUNK)], sems[ci % 2])
    copies[0].wait()
    copies[1].wait()


@functools.partial(
    pl.kernel,
    out_type=jax.ShapeDtypeStruct((B, C, 512, NY), jnp.float32),
    mesh=plsc.VectorSubcoreMesh(
        core_axis_name="c", subcore_axis_name="s",
        num_cores=NC, num_subcores=NS),
    scratch_types=[
        pltpu.VMEM((P,), jnp.int32),
        pltpu.VMEM((P + LANES,), jnp.int32),
        pltpu.VMEM((P + LANES,), jnp.int32),
        pltpu.VMEM((P,), jnp.float32),
        pltpu.VMEM((P,), jnp.float32),
        pltpu.VMEM((ROWS_PER_CHUNK, NY), jnp.float32),
        pltpu.VMEM((ROWS_PER_CHUNK, NY), jnp.float32),
        pltpu.SemaphoreType.DMA,
        pltpu.SemaphoreType.DMA,
        pltpu.SemaphoreType.DMA,
        pltpu.SemaphoreType.DMA,
    ],
    compiler_params=pltpu.CompilerParams(needs_layout_passes=False),
)
def _scatter_planes(flat_hbm, xt_hbm, out_hbm, idx_buf, sel_off, sel_p,
                    x0, x1, chunk0, chunk1, xsem0, xsem1, sem0, sem1):
    _sc_body(flat_hbm, xt_hbm, out_hbm, idx_buf, sel_off, sel_p,
             x0, x1, chunk0, chunk1, xsem0, xsem1, sem0, sem1)


def kernel(x, indices):
    flat = indices[:, :, 0] * 512 + indices[:, :, 1]
    xt = jnp.transpose(x, (0, 2, 1)).reshape(B * C, P)
    return _scatter_planes(flat, xt)
